# baseline (speedup 1.0000x reference)
import functools

import jax
import jax.numpy as jnp
from jax.experimental import pallas as pl
from jax.experimental.pallas import tpu as pltpu


def _gather_tile_kernel(lbl_ref, drop_ref, table_ref, out_ref, sems,
                        *, tile_b: int, cfg_row: int, n_sem: int):
    base = pl.program_id(0) * tile_b
    for r in range(tile_b):
        lbl = lbl_ref[base + r]
        drop = drop_ref[base + r]
        row = jnp.where(drop == 1, cfg_row, lbl)
        row = jnp.clip(row, 0, cfg_row)
        pltpu.make_async_copy(
            table_ref.at[row], out_ref.at[base + r], sems.at[r % n_sem]
        ).start()
    per_sem = tile_b // n_sem
    for j in range(n_sem):
        pltpu.make_async_copy(
            table_ref.at[pl.ds(0, per_sem)], out_ref.at[pl.ds(0, per_sem)],
            sems.at[j],
        ).wait()


def kernel(labels, table, force_drop_ids):
    (B,) = labels.shape
    V, H = table.shape
    cfg_row = V - 1

    labels = labels.astype(jnp.int32)
    force_drop_ids = force_drop_ids.astype(jnp.int32)

    tile_b = 1024
    while B % tile_b != 0:
        tile_b //= 2
    n_b = B // tile_b
    n_sem = 8
    while tile_b % n_sem != 0:
        n_sem //= 2

    table3 = table.reshape(V, 1, H)
    itemsize = jnp.dtype(table.dtype).itemsize

    grid_spec = pltpu.PrefetchScalarGridSpec(
        num_scalar_prefetch=2,
        grid=(n_b,),
        in_specs=[pl.BlockSpec(memory_space=pl.ANY)],
        out_specs=pl.BlockSpec(memory_space=pl.ANY),
        scratch_shapes=[pltpu.SemaphoreType.DMA((n_sem,))],
    )
    out = pl.pallas_call(
        functools.partial(_gather_tile_kernel, tile_b=tile_b, cfg_row=cfg_row,
                          n_sem=n_sem),
        out_shape=jax.ShapeDtypeStruct((B, 1, H), table.dtype),
        grid_spec=grid_spec,
        compiler_params=pltpu.CompilerParams(
            dimension_semantics=("parallel",),
            disable_bounds_checks=True,
        ),
        cost_estimate=pl.CostEstimate(
            flops=0,
            transcendentals=0,
            bytes_accessed=2 * B * H * itemsize + 8 * B),
    )(labels, force_drop_ids, table3)
    return out.reshape(B, H)


# speedup vs baseline: 3.6267x; 3.6267x over previous
import functools

import jax
import jax.numpy as jnp
from jax.experimental import pallas as pl
from jax.experimental.pallas import tpu as pltpu


def _vmem_gather_kernel(lbl_ref, drop_ref, table_ref, out_ref,
                        *, tile_b: int, cfg_row: int):
    base = pl.program_id(0) * tile_b
    for r in range(tile_b):
        lbl = lbl_ref[base + r]
        drop = drop_ref[base + r]
        row = jnp.where(drop == 1, cfg_row, lbl)
        row = jnp.clip(row, 0, cfg_row)
        out_ref[r, 0] = table_ref[row, 0]


def kernel(labels, table, force_drop_ids):
    (B,) = labels.shape
    V, H = table.shape
    cfg_row = V - 1

    labels = labels.astype(jnp.int32)
    force_drop_ids = force_drop_ids.astype(jnp.int32)

    tile_b = 256
    while B % tile_b != 0:
        tile_b //= 2
    n_b = B // tile_b

    table3 = table.reshape(V, 1, H)
    itemsize = jnp.dtype(table.dtype).itemsize

    grid_spec = pltpu.PrefetchScalarGridSpec(
        num_scalar_prefetch=2,
        grid=(n_b,),
        in_specs=[
            pl.BlockSpec((V, 1, H), lambda i, lbl, drp: (0, 0, 0),
                         pipeline_mode=pl.Buffered(1)),
        ],
        out_specs=pl.BlockSpec((tile_b, 1, H), lambda i, lbl, drp: (i, 0, 0)),
    )
    out = pl.pallas_call(
        functools.partial(_vmem_gather_kernel, tile_b=tile_b, cfg_row=cfg_row),
        out_shape=jax.ShapeDtypeStruct((B, 1, H), table.dtype),
        grid_spec=grid_spec,
        compiler_params=pltpu.CompilerParams(
            dimension_semantics=("arbitrary",),
            vmem_limit_bytes=100 * 1024 * 1024,
        ),
        cost_estimate=pl.CostEstimate(
            flops=0,
            transcendentals=0,
            bytes_accessed=(V * H + B * H) * itemsize + 8 * B),
    )(labels, force_drop_ids, table3)
    return out.reshape(B, H)


# speedup vs baseline: 14.3209x; 3.9488x over previous
import functools

import jax
import jax.numpy as jnp
from jax.experimental import pallas as pl
from jax.experimental.pallas import tpu as pltpu


def _vmem_gather_kernel(lbl_ref, drop_ref, table_ref, out_ref,
                        *, tile_b: int, cfg_row: int):
    base = pl.program_id(0) * tile_b
    for r in range(tile_b):
        lbl = lbl_ref[base + r]
        drop = drop_ref[base + r]
        row = jnp.where(drop == 1, cfg_row, lbl)
        row = jnp.clip(row, 0, cfg_row)
        base8 = pl.multiple_of((row >> 3) << 3, 8)
        sub = row & 7
        chunk = table_ref[pl.ds(base8, 8), :]
        rot = pltpu.roll(chunk, (8 - sub) & 7, axis=0)
        out_ref[pl.ds(r, 1), :] = rot[0:1, :]


def kernel(labels, table, force_drop_ids):
    (B,) = labels.shape
    V, H = table.shape
    cfg_row = V - 1

    labels = labels.astype(jnp.int32)
    force_drop_ids = force_drop_ids.astype(jnp.int32)

    tile_b = 256
    while B % tile_b != 0:
        tile_b //= 2
    n_b = B // tile_b
    itemsize = jnp.dtype(table.dtype).itemsize

    grid_spec = pltpu.PrefetchScalarGridSpec(
        num_scalar_prefetch=2,
        grid=(n_b,),
        in_specs=[
            pl.BlockSpec((V, H), lambda i, lbl, drp: (0, 0),
                         pipeline_mode=pl.Buffered(1)),
        ],
        out_specs=pl.BlockSpec((tile_b, H), lambda i, lbl, drp: (i, 0)),
    )
    out = pl.pallas_call(
        functools.partial(_vmem_gather_kernel, tile_b=tile_b, cfg_row=cfg_row),
        out_shape=jax.ShapeDtypeStruct((B, H), table.dtype),
        grid_spec=grid_spec,
        compiler_params=pltpu.CompilerParams(
            dimension_semantics=("arbitrary",),
            vmem_limit_bytes=100 * 1024 * 1024,
            disable_bounds_checks=True,
        ),
        cost_estimate=pl.CostEstimate(
            flops=0,
            transcendentals=0,
            bytes_accessed=(V * H + B * H) * itemsize + 8 * B),
    )(labels, force_drop_ids, table)
    return out
